# Initial kernel scaffold; baseline (speedup 1.0000x reference)
import jax
import jax.numpy as jnp
from jax.experimental import pallas as pl


def kernel(x, edge_index, batch, W1l, b1, W1r, W2l, b2, W2r, W3l, b3, W3r):
    raise NotImplementedError("write your pallas kernel here")


# speedup vs baseline: 3.0932x; 3.0932x over previous
import functools

import jax
import jax.numpy as jnp
from jax import lax
from jax.experimental import pallas as pl
from jax.experimental.pallas import tpu as pltpu
from jax.experimental.pallas import tpu_sc as plsc

D = 128
NP = 10240
EP = 327680
NW = 32
NCH = 80
CH = 128
RPT = NP // 16
NG = 64

_mesh = plsc.VectorSubcoreMesh(core_axis_name="c", subcore_axis_name="s")


def _make_agg(with_counts):
    if with_counts:
        out_type = (jax.ShapeDtypeStruct((2 * NP, D), jnp.float32),
                    jax.ShapeDtypeStruct((2 * NP,), jnp.float32))
    else:
        out_type = jax.ShapeDtypeStruct((2 * NP, D), jnp.float32)
    scratch = [
        pltpu.VMEM((NCH, CH), jnp.int32),
        pltpu.VMEM((NCH, CH), jnp.int32),
        pltpu.VMEM((CH, D), jnp.float32),
        pltpu.VMEM_SHARED((NP, D), jnp.float32),
        pltpu.SemaphoreType.DMA,
    ]
    if with_counts:
        scratch += [pltpu.VMEM((CH,), jnp.float32),
                    pltpu.VMEM_SHARED((NP,), jnp.float32)]

    def body(*refs):
        if with_counts:
            (h_hbm, src_hbm, dst_hbm, zrow_hbm, zcnt_hbm,
             aggp, cntp, src_v, dst_v, rows_v, acc_sh, sem,
             ones_v, cnt_sh) = refs
        else:
            (h_hbm, src_hbm, dst_hbm, zrow_hbm,
             aggp, src_v, dst_v, rows_v, acc_sh, sem) = refs
        cid = lax.axis_index("c")
        sid = lax.axis_index("s")
        wid = sid * 2 + cid
        pltpu.sync_copy(zrow_hbm, acc_sh.at[pl.ds(sid * RPT, RPT)])
        pltpu.sync_copy(src_hbm.at[wid], src_v)
        pltpu.sync_copy(dst_hbm.at[wid], dst_v)
        if with_counts:
            pltpu.sync_copy(zcnt_hbm, cnt_sh.at[pl.ds(sid * (NP // 16),
                                                      NP // 16)])
            for g in range(CH // 16):
                ones_v[pl.ds(g * 16, 16)] = jnp.ones((16,), jnp.float32)
        plsc.subcore_barrier()

        def chunk(j, carry):
            pltpu.async_copy(h_hbm.at[src_v.at[j]], rows_v, sem).wait()
            pltpu.sync_copy(rows_v, acc_sh.at[dst_v.at[j]], add=True)
            if with_counts:
                pltpu.sync_copy(ones_v, cnt_sh.at[dst_v.at[j]], add=True)
            return carry

        lax.fori_loop(0, NCH, chunk, 0)
        plsc.subcore_barrier()
        pltpu.sync_copy(acc_sh.at[pl.ds(sid * RPT, RPT)],
                        aggp.at[pl.ds(cid * NP + sid * RPT, RPT)])
        if with_counts:
            pltpu.sync_copy(cnt_sh.at[pl.ds(sid * (NP // 16), NP // 16)],
                            cntp.at[pl.ds(cid * NP + sid * (NP // 16),
                                          NP // 16)])

    return pl.kernel(body, out_type=out_type, mesh=_mesh,
                     scratch_types=scratch)


_agg_counts = _make_agg(True)
_agg = _make_agg(False)


def _layer_body(aggp_ref, cntp_ref, h_ref, wl_ref, b_ref, wr_ref, o_ref):
    aggs = aggp_ref[0] + aggp_ref[1]
    cnt = jnp.sum(cntp_ref[...], axis=0)
    inv = 1.0 / jnp.maximum(cnt, 1.0)
    a = aggs * inv
    out = (jnp.dot(a, wl_ref[...], preferred_element_type=jnp.float32)
           + b_ref[...]
           + jnp.dot(h_ref[...], wr_ref[...], preferred_element_type=jnp.float32))
    o_ref[...] = jnp.maximum(out, 0.0)


def _tc_layer(aggp, cntp, h, wlT, b, wrT, block=512):
    grid = (NP // block,)
    return pl.pallas_call(
        _layer_body,
        grid=grid,
        in_specs=[
            pl.BlockSpec((2, block, D), lambda i: (0, i, 0)),
            pl.BlockSpec((2, block, 1), lambda i: (0, i, 0)),
            pl.BlockSpec((block, D), lambda i: (i, 0)),
            pl.BlockSpec((D, D), lambda i: (0, 0)),
            pl.BlockSpec((1, D), lambda i: (0, 0)),
            pl.BlockSpec((D, D), lambda i: (0, 0)),
        ],
        out_specs=pl.BlockSpec((block, D), lambda i: (i, 0)),
        out_shape=jax.ShapeDtypeStruct((NP, D), jnp.float32),
    )(aggp, cntp, h, wlT, b, wrT)


def _pool_body(nblk, aggp_ref, cntp_ref, h_ref, wl_ref, b_ref, wr_ref,
               batch_ref, o_ref, cg_ref):
    i = pl.program_id(0)

    @pl.when(i == 0)
    def _():
        o_ref[...] = jnp.zeros_like(o_ref)
        cg_ref[...] = jnp.zeros_like(cg_ref)

    aggs = aggp_ref[0] + aggp_ref[1]
    cnt = jnp.sum(cntp_ref[...], axis=0)
    inv = 1.0 / jnp.maximum(cnt, 1.0)
    a = aggs * inv
    h3 = (jnp.dot(a, wl_ref[...], preferred_element_type=jnp.float32)
          + b_ref[...]
          + jnp.dot(h_ref[...], wr_ref[...], preferred_element_type=jnp.float32))
    h3 = jnp.maximum(h3, 0.0)
    gid = lax.broadcasted_iota(jnp.int32, (1, NG), 1)
    oh = (batch_ref[...] == gid).astype(jnp.float32)
    dn = (((0,), (0,)), ((), ()))
    o_ref[...] += lax.dot_general(oh, h3, dn, preferred_element_type=jnp.float32)
    cg_ref[...] += lax.dot_general(oh, jnp.ones_like(h3), dn,
                                   preferred_element_type=jnp.float32)

    @pl.when(i == nblk - 1)
    def _():
        o_ref[...] = o_ref[...] / jnp.maximum(cg_ref[...], 1.0)


def _tc_pool(aggp, cntp, h, wlT, b, wrT, batch_p, block=512):
    nblk = NP // block
    return pl.pallas_call(
        functools.partial(_pool_body, nblk),
        grid=(nblk,),
        in_specs=[
            pl.BlockSpec((2, block, D), lambda i: (0, i, 0)),
            pl.BlockSpec((2, block, 1), lambda i: (0, i, 0)),
            pl.BlockSpec((block, D), lambda i: (i, 0)),
            pl.BlockSpec((D, D), lambda i: (0, 0)),
            pl.BlockSpec((1, D), lambda i: (0, 0)),
            pl.BlockSpec((D, D), lambda i: (0, 0)),
            pl.BlockSpec((block, 1), lambda i: (i, 0)),
        ],
        out_specs=pl.BlockSpec((NG, D), lambda i: (0, 0)),
        out_shape=jax.ShapeDtypeStruct((NG, D), jnp.float32),
        scratch_shapes=[pltpu.VMEM((NG, D), jnp.float32)],
    )(aggp, cntp, h, wlT, b, wrT, batch_p)


def kernel(x, edge_index, batch, W1l, b1, W1r, W2l, b2, W2r, W3l, b3, W3r):
    n, d = x.shape
    e = edge_index.shape[1]
    src = edge_index[0].astype(jnp.int32)
    dst = edge_index[1].astype(jnp.int32)
    x_pad = jnp.concatenate([x, jnp.zeros((NP - n, d), x.dtype)], axis=0)
    src_r = jnp.concatenate([src, jnp.zeros((EP - e,), jnp.int32)]
                            ).reshape(NW, NCH, CH)
    dst_r = jnp.concatenate([dst, jnp.full((EP - e,), NP - 1, jnp.int32)]
                            ).reshape(NW, NCH, CH)
    batch_p = jnp.concatenate([batch.astype(jnp.int32),
                               jnp.full((NP - n,), NG, jnp.int32)]
                              ).reshape(NP, 1)
    zrow = jnp.zeros((RPT, D), jnp.float32)
    zcnt = jnp.zeros((NP // 16,), jnp.float32)

    aggp1, cntp = _agg_counts(x_pad, src_r, dst_r, zrow, zcnt)
    aggp1 = aggp1.reshape(2, NP, D)
    cntp = cntp.reshape(2, NP, 1)
    h1 = _tc_layer(aggp1, cntp, x_pad, W1l.T, b1.reshape(1, D), W1r.T)
    aggp2 = _agg(h1, src_r, dst_r, zrow).reshape(2, NP, D)
    h2 = _tc_layer(aggp2, cntp, h1, W2l.T, b2.reshape(1, D), W2r.T)
    aggp3 = _agg(h2, src_r, dst_r, zrow).reshape(2, NP, D)
    return _tc_pool(aggp3, cntp, h2, W3l.T, b3.reshape(1, D), W3r.T, batch_p)
